# baseline (speedup 1.0000x reference)
import functools

import jax
import jax.numpy as jnp
from jax import lax
from jax.experimental import pallas as pl
from jax.experimental.pallas import tpu as pltpu
from jax.experimental.pallas import tpu_sc as plsc

_B = 16384
_D = 32
_LP = 128
_QU = 250000
_QI = 25000
_L = 16
_NC = 2
_NS = 16
_NW = _NC * _NS
_BPW = _B // _NW
_C = 128
_NCH = _BPW // _C

_mesh = plsc.VectorSubcoreMesh(core_axis_name="c", subcore_axis_name="s")


def _quarter(v, q):
    one = jnp.ones((_L,), jnp.int32)
    zero = jnp.zeros((_L,), jnp.int32)
    c = jnp.where(v >= q, one, zero)
    c = c + jnp.where(v >= 2 * q, one, zero)
    c = c + jnp.where(v >= 3 * q, one, zero)
    return c


@functools.partial(
    pl.kernel,
    out_type=jax.ShapeDtypeStruct((_B,), jnp.float32),
    mesh=_mesh,
    scratch_types=[
        pltpu.VMEM((_BPW,), jnp.int32),
        pltpu.VMEM((_BPW,), jnp.int32),
        pltpu.VMEM((_C, _LP), jnp.float32),
        pltpu.VMEM((_C, _LP), jnp.float32),
        pltpu.VMEM((_C, _LP), jnp.float32),
        pltpu.VMEM((_C, _LP), jnp.float32),
        pltpu.VMEM((_BPW,), jnp.float32),
        pltpu.SemaphoreType.DMA,
        pltpu.SemaphoreType.DMA,
    ],
    compiler_params=pltpu.CompilerParams(
        needs_layout_passes=False, use_tc_tiling_on_sc=True,
        disable_bounds_checks=True, disable_semaphore_checks=True),
)
def _mf_sc(user_hbm, item_hbm, ufp_hbm, ifp_hbm, out_hbm,
           uidx_v, iidx_v,
           ubuf0, ubuf1, ibuf0, ibuf1, out_v, sem0, sem1):
    wid = lax.axis_index("s") * _NC + lax.axis_index("c")
    base = wid * _BPW

    pltpu.sync_copy(user_hbm.at[pl.ds(base, _BPW)], uidx_v)
    pltpu.sync_copy(item_hbm.at[pl.ds(base, _BPW)], iidx_v)

    ub = (ubuf0, ubuf1)
    ib = (ibuf0, ibuf1)
    sems = (sem0, sem1)

    def _fire(c, p):
        def body(v, carry):
            u0 = uidx_v[pl.ds(c * _C + v * _L, _L)] - 1
            i0 = iidx_v[pl.ds(c * _C + v * _L, _L)] - 1
            urow = u0 - _quarter(u0, _QU) * _QU
            irow = i0 - _quarter(i0, _QI) * _QI
            for j in range(_L):
                k = v * _L + j
                pltpu.async_copy(
                    ufp_hbm.at[pl.ds(urow[j], 1)], ub[p].at[pl.ds(k, 1)], sems[p])
                pltpu.async_copy(
                    ifp_hbm.at[pl.ds(irow[j], 1)], ib[p].at[pl.ds(k, 1)], sems[p])
            return carry
        lax.fori_loop(0, _C // _L, body, 0)

    def _drain(p):
        pltpu.make_async_copy(ufp_hbm.at[pl.ds(0, _C)], ub[p], sems[p]).wait()
        pltpu.make_async_copy(ifp_hbm.at[pl.ds(0, _C)], ib[p], sems[p]).wait()

    def _compute(c, p):
        u, it = ub[p], ib[p]

        def grp(g, carry):
            sl = pl.ds(c * _C + g * _L, _L)
            row = g * _L + lax.iota(jnp.int32, _L)
            ucol0 = lax.shift_left(_quarter(uidx_v[sl] - 1, _QU), 5)
            icol0 = lax.shift_left(_quarter(iidx_v[sl] - 1, _QI), 5)
            acc = jnp.zeros((_L,), jnp.float32)
            for d in range(_D):
                gu = plsc.load_gather(u, [row, ucol0 + d])
                gi = plsc.load_gather(it, [row, icol0 + d])
                acc = acc + gu * gi
            out_v[pl.ds(c * _C + g * _L, _L)] = acc * 5.0
            return carry

        lax.fori_loop(0, _C // _L, grp, 0)

    _fire(0, 0)
    for c in range(_NCH):
        p = c % 2
        if c + 1 < _NCH:
            _fire(c + 1, (c + 1) % 2)
        _drain(p)
        _compute(c, p)

    pltpu.sync_copy(out_v, out_hbm.at[pl.ds(base, _BPW)])


def kernel(user, item, user_factors, item_factors):
    ufp = jnp.concatenate(
        [user_factors[q * _QU:(q + 1) * _QU] for q in range(4)], axis=1)
    ifp = jnp.concatenate(
        [item_factors[q * _QI:(q + 1) * _QI] for q in range(4)], axis=1)
    return _mf_sc(user, item, ufp, ifp)


# speedup vs baseline: 1.9081x; 1.9081x over previous
import functools

import jax
import jax.numpy as jnp
from jax import lax
from jax.experimental import pallas as pl
from jax.experimental.pallas import tpu as pltpu
from jax.experimental.pallas import tpu_sc as plsc

_B = 16384
_D = 32
_L = 16
_NC = 2
_NS = 16
_NW = _NC * _NS
_BPW = _B // _NW
_C = 128
_NCH = _BPW // _C

_mesh = plsc.VectorSubcoreMesh(core_axis_name="c", subcore_axis_name="s")


@functools.partial(
    pl.kernel,
    out_type=jax.ShapeDtypeStruct((_B,), jnp.float32),
    mesh=_mesh,
    scratch_types=[
        pltpu.VMEM((_BPW,), jnp.int32),
        pltpu.VMEM((_BPW,), jnp.int32),
        pltpu.VMEM((_C, _D), jnp.float32),
        pltpu.VMEM((_C, _D), jnp.float32),
        pltpu.VMEM((_C, _D), jnp.float32),
        pltpu.VMEM((_C, _D), jnp.float32),
        pltpu.VMEM((_BPW,), jnp.float32),
        pltpu.SemaphoreType.DMA,
        pltpu.SemaphoreType.DMA,
    ],
    compiler_params=pltpu.CompilerParams(
        needs_layout_passes=False, use_tc_tiling_on_sc=True,
        disable_bounds_checks=True, disable_semaphore_checks=True),
)
def _mf_sc(user_hbm, item_hbm, uf_hbm, if_hbm, out_hbm,
           uidx_v, iidx_v,
           ubuf0, ubuf1, ibuf0, ibuf1, out_v, sem0, sem1):
    wid = lax.axis_index("s") * _NC + lax.axis_index("c")
    base = wid * _BPW

    pltpu.sync_copy(user_hbm.at[pl.ds(base, _BPW)], uidx_v)
    pltpu.sync_copy(item_hbm.at[pl.ds(base, _BPW)], iidx_v)

    ub = (ubuf0, ubuf1)
    ib = (ibuf0, ibuf1)
    sems = (sem0, sem1)

    def _fire(c, p):
        def body(v, carry):
            uvec = uidx_v[pl.ds(c * _C + v * _L, _L)] - 1
            ivec = iidx_v[pl.ds(c * _C + v * _L, _L)] - 1
            for j in range(_L):
                k = v * _L + j
                pltpu.async_copy(
                    uf_hbm.at[pl.ds(uvec[j], 1)], ub[p].at[pl.ds(k, 1)], sems[p])
                pltpu.async_copy(
                    if_hbm.at[pl.ds(ivec[j], 1)], ib[p].at[pl.ds(k, 1)], sems[p])
            return carry
        lax.fori_loop(0, _C // _L, body, 0)

    def _drain(p):
        pltpu.make_async_copy(uf_hbm.at[pl.ds(0, _C)], ub[p], sems[p]).wait()
        pltpu.make_async_copy(if_hbm.at[pl.ds(0, _C)], ib[p], sems[p]).wait()

    def _compute(c, p):
        u, it = ub[p], ib[p]

        def grp(g, carry):
            row = g * _L + lax.iota(jnp.int32, _L)
            acc = jnp.zeros((_L,), jnp.float32)
            for d in range(_D):
                col = jnp.full((_L,), d, jnp.int32)
                acc = acc + plsc.load_gather(u, [row, col]) * plsc.load_gather(it, [row, col])
            out_v[pl.ds(c * _C + g * _L, _L)] = acc * 5.0
            return carry

        lax.fori_loop(0, _C // _L, grp, 0)

    _fire(0, 0)
    for c in range(_NCH):
        p = c % 2
        if c + 1 < _NCH:
            _fire(c + 1, (c + 1) % 2)
        _drain(p)
        _compute(c, p)

    pltpu.sync_copy(out_v, out_hbm.at[pl.ds(base, _BPW)])


def kernel(user, item, user_factors, item_factors):
    return _mf_sc(user, item, user_factors, item_factors)
